# Initial kernel scaffold; baseline (speedup 1.0000x reference)
import jax
import jax.numpy as jnp
from jax.experimental import pallas as pl


def kernel(h, edge_index, coord, edge_attr, W_e1, b_e1, W_e2, b_e2, W_n1, b_n1, W_n2, b_n2, W_c1, b_c1, W_c2):
    raise NotImplementedError("write your pallas kernel here")


# speedup vs baseline: 3.8345x; 3.8345x over previous
import functools

import jax
import jax.numpy as jnp
from jax import lax
from jax.experimental import pallas as pl
from jax.experimental.pallas import tpu as pltpu
from jax.experimental.pallas import tpu_sc as plsc

_C = 128
_BE = 2000
_BN = 1000
_NW = 32
_ZR = 125


def _sc_gather(h, coordp, row2d, col2d):
    n, d = h.shape
    nchunk = row2d.shape[0]
    e = nchunk * _C
    iters = (nchunk + _NW - 1) // _NW
    mesh = plsc.VectorSubcoreMesh(core_axis_name="c", subcore_axis_name="s")
    f32 = jnp.float32

    @functools.partial(
        pl.kernel,
        out_type=(
            jax.ShapeDtypeStruct((e, d), f32),
            jax.ShapeDtypeStruct((e, d), f32),
            jax.ShapeDtypeStruct((e, 16), f32),
            jax.ShapeDtypeStruct((e, 16), f32),
        ),
        mesh=mesh,
        scratch_types=[
            pltpu.VMEM((_C,), jnp.int32),
            pltpu.VMEM((_C,), jnp.int32),
            pltpu.VMEM((_C, d), f32),
            pltpu.VMEM((_C, d), f32),
            pltpu.VMEM((_C, 16), f32),
            pltpu.VMEM((_C, 16), f32),
            pltpu.SemaphoreType.DMA,
            pltpu.SemaphoreType.DMA,
            pltpu.SemaphoreType.DMA,
            pltpu.SemaphoreType.DMA,
        ],
        compiler_params=pltpu.CompilerParams(use_tc_tiling_on_sc=False),
    )
    def gk(h_hbm, cp_hbm, r2_hbm, c2_hbm, hr_o, hc_o, cr_o, cc_o,
           ir, ic, bhr, bhc, bcr, bcc, s1, s2, s3, s4):
        wid = lax.axis_index("s") * 2 + lax.axis_index("c")

        def body(i, carry):
            ch = wid + i * _NW

            @pl.when(ch < nchunk)
            def _():
                pltpu.sync_copy(r2_hbm.at[ch], ir)
                pltpu.sync_copy(c2_hbm.at[ch], ic)
                d1 = pltpu.async_copy(h_hbm.at[ir], bhr, s1)
                d2 = pltpu.async_copy(h_hbm.at[ic], bhc, s2)
                d3 = pltpu.async_copy(cp_hbm.at[ir], bcr, s3)
                d4 = pltpu.async_copy(cp_hbm.at[ic], bcc, s4)
                d1.wait()
                d2.wait()
                d3.wait()
                d4.wait()
                base = ch * _C
                pltpu.sync_copy(bhr, hr_o.at[pl.ds(base, _C)])
                pltpu.sync_copy(bhc, hc_o.at[pl.ds(base, _C)])
                pltpu.sync_copy(bcr, cr_o.at[pl.ds(base, _C)])
                pltpu.sync_copy(bcc, cc_o.at[pl.ds(base, _C)])

            return carry

        lax.fori_loop(0, iters, body, 0)

    return gk(h, coordp, row2d, col2d)


def _sc_scatter(ef, tc, row2d, n):
    e, d = ef.shape
    nchunk = row2d.shape[0]
    iters = (nchunk + _NW - 1) // _NW
    npt = n // 16
    mesh = plsc.VectorSubcoreMesh(core_axis_name="c", subcore_axis_name="s")
    f32 = jnp.float32

    @functools.partial(
        pl.kernel,
        out_type=(
            jax.ShapeDtypeStruct((n, d), f32),
            jax.ShapeDtypeStruct((n, d), f32),
            jax.ShapeDtypeStruct((n, 16), f32),
            jax.ShapeDtypeStruct((n, 16), f32),
        ),
        mesh=mesh,
        scratch_types=[
            pltpu.VMEM((_C,), jnp.int32),
            pltpu.VMEM((_C, d), f32),
            pltpu.VMEM((_C, 16), f32),
            pltpu.VMEM((_ZR, d), f32),
            pltpu.VMEM((_ZR, 16), f32),
            pltpu.VMEM_SHARED((n, d), f32),
            pltpu.VMEM_SHARED((n, 16), f32),
        ],
        compiler_params=pltpu.CompilerParams(use_tc_tiling_on_sc=False),
    )
    def sk(ef_hbm, tc_hbm, r2_hbm, an0, an1, at0, at1,
           idx, bef, btc, zb1, zb2, accn, acct):
        c = lax.axis_index("c")
        s = lax.axis_index("s")
        wid = s * 2 + c

        def zrow(i, carry):
            for j in range(d // 16):
                zb1[i, pl.ds(j * 16, 16)] = jnp.zeros((16,), f32)
            zb2[i, pl.ds(0, 16)] = jnp.zeros((16,), f32)
            return carry

        lax.fori_loop(0, _ZR, zrow, 0)
        for k in range(npt // _ZR):
            pltpu.sync_copy(zb1, accn.at[pl.ds(s * npt + k * _ZR, _ZR)])
            pltpu.sync_copy(zb2, acct.at[pl.ds(s * npt + k * _ZR, _ZR)])
        plsc.subcore_barrier()

        def body(i, carry):
            ch = wid + i * _NW

            @pl.when(ch < nchunk)
            def _():
                pltpu.sync_copy(r2_hbm.at[ch], idx)
                base = ch * _C
                pltpu.sync_copy(ef_hbm.at[pl.ds(base, _C)], bef)
                pltpu.sync_copy(tc_hbm.at[pl.ds(base, _C)], btc)
                pltpu.sync_copy(bef, accn.at[idx], add=True)
                pltpu.sync_copy(btc, acct.at[idx], add=True)

            return carry

        lax.fori_loop(0, iters, body, 0)
        plsc.subcore_barrier()

        @pl.when(c == 0)
        def _():
            pltpu.sync_copy(accn.at[pl.ds(s * npt, npt)], an0.at[pl.ds(s * npt, npt)])
            pltpu.sync_copy(acct.at[pl.ds(s * npt, npt)], at0.at[pl.ds(s * npt, npt)])

        @pl.when(c == 1)
        def _():
            pltpu.sync_copy(accn.at[pl.ds(s * npt, npt)], an1.at[pl.ds(s * npt, npt)])
            pltpu.sync_copy(acct.at[pl.ds(s * npt, npt)], at1.at[pl.ds(s * npt, npt)])

    return sk(ef, tc, row2d)


def _tc_edge(hr, hc, ea, cr, cc, W1h, W1c, w1r, W1a, b1, W2, b2, Wc1, bc1, wc2):
    e, d = hr.shape
    he = W2.shape[1]
    f32 = jnp.float32

    def body(hr_r, hc_r, ea_r, cr_r, cc_r, W1h_r, W1c_r, w1r_r, W1a_r, b1_r,
             W2_r, b2_r, Wc1_r, bc1_r, wc2_r, ef_o, tc_o):
        dif = cr_r[...] - cc_r[...]
        radial = jnp.sum(dif * dif, axis=1, keepdims=True)
        x = (jnp.dot(hr_r[...], W1h_r[...], preferred_element_type=f32)
             + jnp.dot(hc_r[...], W1c_r[...], preferred_element_type=f32)
             + jnp.dot(ea_r[...], W1a_r[...], preferred_element_type=f32)
             + radial * w1r_r[...]
             + b1_r[...])
        x = jnp.maximum(x, 0.0)
        ef = jnp.maximum(jnp.dot(x, W2_r[...], preferred_element_type=f32)
                         + b2_r[...], 0.0)
        c1 = jnp.maximum(jnp.dot(ef, Wc1_r[...], preferred_element_type=f32)
                         + bc1_r[...], 0.0)
        scal = jnp.sum(c1 * wc2_r[...], axis=1, keepdims=True)
        tr = jnp.clip(dif * scal, -100.0, 100.0)
        lane = lax.broadcasted_iota(jnp.int32, (_BE, 16), 1)
        tc_o[...] = tr + jnp.where(lane == 3, 1.0, 0.0)
        ef_o[...] = ef

    wspec = pl.BlockSpec((d, he), lambda i: (0, 0))
    vspec = pl.BlockSpec((1, he), lambda i: (0, 0))
    return pl.pallas_call(
        body,
        grid=(e // _BE,),
        in_specs=[
            pl.BlockSpec((_BE, d), lambda i: (i, 0)),
            pl.BlockSpec((_BE, d), lambda i: (i, 0)),
            pl.BlockSpec((_BE, d), lambda i: (i, 0)),
            pl.BlockSpec((_BE, 16), lambda i: (i, 0)),
            pl.BlockSpec((_BE, 16), lambda i: (i, 0)),
            wspec, wspec, vspec, wspec, vspec,
            wspec, vspec, wspec, vspec, vspec,
        ],
        out_specs=[
            pl.BlockSpec((_BE, he), lambda i: (i, 0)),
            pl.BlockSpec((_BE, 16), lambda i: (i, 0)),
        ],
        out_shape=[
            jax.ShapeDtypeStruct((e, he), f32),
            jax.ShapeDtypeStruct((e, 16), f32),
        ],
    )(hr, hc, ea, cr, cc, W1h, W1c, w1r, W1a, b1, W2, b2, Wc1, bc1, wc2)


def _tc_node(h, coordp, an0, an1, at0, at1, Wn1h, Wn1a, bn1, Wn2, bn2):
    n, d = h.shape
    f32 = jnp.float32

    def body(h_r, cp_r, an0_r, an1_r, at0_r, at1_r, Wn1h_r, Wn1a_r, bn1_r,
             Wn2_r, bn2_r, ho_o, co_o):
        aggn = an0_r[...] + an1_r[...]
        aggt = at0_r[...] + at1_r[...]
        hid = jnp.maximum(
            jnp.dot(h_r[...], Wn1h_r[...], preferred_element_type=f32)
            + jnp.dot(aggn, Wn1a_r[...], preferred_element_type=f32)
            + bn1_r[...], 0.0)
        ho_o[...] = (jnp.dot(hid, Wn2_r[...], preferred_element_type=f32)
                     + bn2_r[...] + h_r[...])
        lane = lax.broadcasted_iota(jnp.int32, (_BN, 16), 1)
        cnt = jnp.sum(jnp.where(lane == 3, aggt, 0.0), axis=1, keepdims=True)
        cnt = jnp.maximum(cnt, 1.0)
        co_o[...] = cp_r[...] + jnp.where(lane < 3, aggt / cnt, 0.0)

    wspec = pl.BlockSpec((d, d), lambda i: (0, 0))
    vspec = pl.BlockSpec((1, d), lambda i: (0, 0))
    return pl.pallas_call(
        body,
        grid=(n // _BN,),
        in_specs=[
            pl.BlockSpec((_BN, d), lambda i: (i, 0)),
            pl.BlockSpec((_BN, 16), lambda i: (i, 0)),
            pl.BlockSpec((_BN, d), lambda i: (i, 0)),
            pl.BlockSpec((_BN, d), lambda i: (i, 0)),
            pl.BlockSpec((_BN, 16), lambda i: (i, 0)),
            pl.BlockSpec((_BN, 16), lambda i: (i, 0)),
            wspec, wspec, vspec, wspec, vspec,
        ],
        out_specs=[
            pl.BlockSpec((_BN, d), lambda i: (i, 0)),
            pl.BlockSpec((_BN, 16), lambda i: (i, 0)),
        ],
        out_shape=[
            jax.ShapeDtypeStruct((n, d), f32),
            jax.ShapeDtypeStruct((n, 16), f32),
        ],
    )(h, coordp, an0, an1, at0, at1, Wn1h, Wn1a, bn1, Wn2, bn2)


def kernel(h, edge_index, coord, edge_attr, W_e1, b_e1, W_e2, b_e2,
           W_n1, b_n1, W_n2, b_n2, W_c1, b_c1, W_c2):
    n, d = h.shape
    e = edge_index.shape[1]
    f32 = jnp.float32

    row2d = edge_index[0].reshape(e // _C, _C)
    col2d = edge_index[1].reshape(e // _C, _C)
    coordp = jnp.concatenate(
        [coord, jnp.zeros((n, 13), f32)], axis=1)

    hr, hc, cr, cc = _sc_gather(h, coordp, row2d, col2d)

    W1h = W_e1[:d]
    W1c = W_e1[d:2 * d]
    w1r = W_e1[2 * d:2 * d + 1]
    W1a = W_e1[2 * d + 1:]
    ef, tc = _tc_edge(hr, hc, edge_attr, cr, cc,
                      W1h, W1c, w1r, W1a, b_e1.reshape(1, -1),
                      W_e2, b_e2.reshape(1, -1),
                      W_c1, b_c1.reshape(1, -1), W_c2.reshape(1, -1))

    an0, an1, at0, at1 = _sc_scatter(ef, tc, row2d, n)

    h_out, co = _tc_node(h, coordp, an0, an1, at0, at1,
                         W_n1[:d], W_n1[d:], b_n1.reshape(1, -1),
                         W_n2, b_n2.reshape(1, -1))
    coord_out = co[:, :3].reshape(n, 3, 1)
    return (h_out, coord_out, edge_attr)
